# baseline (speedup 1.0000x reference)
import functools
import math

import jax
import jax.numpy as jnp
from jax import lax
from jax.experimental import pallas as pl
from jax.experimental.pallas import tpu as pltpu
from jax.experimental.pallas import tpu_sc as plsc

VOCAB = 1000000
EMBED_DIM = 64
HARMONIC_BASES = 16

_B, _S = 1024, 50
_T = _B * _S
_NC, _NS = 2, 16
_NW = _NC * _NS
_TPW = _T // _NW
_EPW = _TPW * HARMONIC_BASES
_PR = _T // 8


def _sc_gather(ilist, flat_a, flat_p):
    mesh = plsc.VectorSubcoreMesh(core_axis_name="c", subcore_axis_name="s")

    @functools.partial(
        pl.kernel,
        out_type=(
            jax.ShapeDtypeStruct((_T * HARMONIC_BASES,), jnp.float32),
            jax.ShapeDtypeStruct((_T * HARMONIC_BASES,), jnp.float32),
        ),
        mesh=mesh,
        scratch_types=[
            pltpu.VMEM((_EPW,), jnp.int32),
            pltpu.VMEM((_EPW,), jnp.float32),
            pltpu.VMEM((_EPW,), jnp.float32),
            pltpu.SemaphoreType.DMA,
        ],
        compiler_params=pltpu.CompilerParams(use_tc_tiling_on_sc=False),
    )
    def gather_kernel(ilist_hbm, a_hbm, p_hbm, a_out, p_out,
                      ilist_v, vals_a, vals_p, sem):
        wid = lax.axis_index("s") * _NC + lax.axis_index("c")
        base = wid * _EPW
        pltpu.sync_copy(ilist_hbm.at[pl.ds(base, _EPW)], ilist_v)
        cp_a = pltpu.async_copy(a_hbm.at[ilist_v], vals_a, sem)
        cp_p = pltpu.async_copy(p_hbm.at[ilist_v], vals_p, sem)
        cp_a.wait()
        cp_p.wait()
        pltpu.sync_copy(vals_a, a_out.at[pl.ds(base, _EPW)])
        pltpu.sync_copy(vals_p, p_out.at[pl.ds(base, _EPW)])

    return gather_kernel(ilist, flat_a, flat_p)


_BR = 320


def _tc_body(amp_ref, phase_ref, sb_ref, cb_ref, out_ref):
    a = amp_ref[...]
    p = phase_ref[...]
    w = a * jnp.cos(p)
    z = a * jnp.sin(p)
    out_ref[...] = (
        jnp.dot(w, sb_ref[...], preferred_element_type=jnp.float32)
        + jnp.dot(z, cb_ref[...], preferred_element_type=jnp.float32)
    )


def _tc_synth(amp_p, phase_p, sb, cb):
    grid = (_PR // _BR,)
    return pl.pallas_call(
        _tc_body,
        grid=grid,
        in_specs=[
            pl.BlockSpec((_BR, 128), lambda i: (i, 0)),
            pl.BlockSpec((_BR, 128), lambda i: (i, 0)),
            pl.BlockSpec((128, 8 * EMBED_DIM), lambda i: (0, 0)),
            pl.BlockSpec((128, 8 * EMBED_DIM), lambda i: (0, 0)),
        ],
        out_specs=pl.BlockSpec((_BR, 8 * EMBED_DIM), lambda i: (i, 0)),
        out_shape=jax.ShapeDtypeStruct((_PR, 8 * EMBED_DIM), jnp.float32),
    )(amp_p, phase_p, sb, cb)


def kernel(x, frequency_amplitudes, frequency_phases, frequencies):
    idx = x.reshape(_T).astype(jnp.int32)
    harm = jnp.tile(jnp.arange(HARMONIC_BASES, dtype=jnp.int32) * VOCAB, _T)
    ilist = jnp.repeat(idx, HARMONIC_BASES) + harm
    flat_a = frequency_amplitudes.T.reshape(VOCAB * HARMONIC_BASES)
    flat_p = frequency_phases.T.reshape(VOCAB * HARMONIC_BASES)

    a_flat, p_flat = _sc_gather(ilist, flat_a, flat_p)
    amp_p = a_flat.reshape(_PR, 128)
    phase_p = p_flat.reshape(_PR, 128)

    t = jnp.linspace(0.0, 1.0, EMBED_DIM, dtype=jnp.float32)
    theta = (2.0 * math.pi) * frequencies[:, None] * t[None, :]
    eye8 = jnp.eye(8, dtype=jnp.float32)
    sb = jnp.kron(eye8, jnp.sin(theta))
    cb = jnp.kron(eye8, jnp.cos(theta))

    out = _tc_synth(amp_p, phase_p, sb, cb)
    return out.reshape(_B, _S, EMBED_DIM)


# speedup vs baseline: 6.7213x; 6.7213x over previous
import functools
import math

import jax
import jax.numpy as jnp
from jax import lax
from jax.experimental import pallas as pl
from jax.experimental.pallas import tpu as pltpu
from jax.experimental.pallas import tpu_sc as plsc

VOCAB = 1000000
EMBED_DIM = 64
HARMONIC_BASES = 16

_B, _S = 1024, 50
_T = _B * _S
_NC, _NS = 2, 16
_NW = _NC * _NS
_TPW = _T // _NW
_EPW = _TPW * HARMONIC_BASES
_PR = _T // 8


def _sc_gather(ilist, flat_a, flat_p):
    mesh = plsc.VectorSubcoreMesh(core_axis_name="c", subcore_axis_name="s")

    @functools.partial(
        pl.kernel,
        out_type=(
            jax.ShapeDtypeStruct((_T * HARMONIC_BASES,), jnp.float32),
            jax.ShapeDtypeStruct((_T * HARMONIC_BASES,), jnp.float32),
        ),
        name="sc_spectral_gather",
        mesh=mesh,
        scratch_types=[
            pltpu.VMEM((_EPW,), jnp.int32),
            pltpu.VMEM((_EPW,), jnp.float32),
            pltpu.VMEM((_EPW,), jnp.float32),
            pltpu.SemaphoreType.DMA,
        ],
        compiler_params=pltpu.CompilerParams(use_tc_tiling_on_sc=False),
    )
    def gather_kernel(ilist_hbm, a_hbm, p_hbm, a_out, p_out,
                      ilist_v, vals_a, vals_p, sem):
        wid = lax.axis_index("s") * _NC + lax.axis_index("c")
        base = wid * _EPW
        pltpu.sync_copy(ilist_hbm.at[pl.ds(base, _EPW)], ilist_v)
        cp_a = pltpu.async_copy(a_hbm.at[ilist_v], vals_a, sem)
        cp_p = pltpu.async_copy(p_hbm.at[ilist_v], vals_p, sem)
        cp_a.wait()
        cp_p.wait()
        pltpu.sync_copy(vals_a, a_out.at[pl.ds(base, _EPW)])
        pltpu.sync_copy(vals_p, p_out.at[pl.ds(base, _EPW)])

    return gather_kernel(ilist, flat_a, flat_p)


_BR = 320


def _tc_body(amp_ref, phase_ref, sb_ref, cb_ref, out_ref):
    a = amp_ref[...]
    p = phase_ref[...]
    w = a * jnp.cos(p)
    z = a * jnp.sin(p)
    out_ref[...] = (
        jnp.dot(w, sb_ref[...], preferred_element_type=jnp.float32)
        + jnp.dot(z, cb_ref[...], preferred_element_type=jnp.float32)
    )


def _tc_synth(amp_p, phase_p, sb, cb):
    grid = (_PR // _BR,)
    return pl.pallas_call(
        _tc_body,
        grid=grid,
        in_specs=[
            pl.BlockSpec((_BR, 128), lambda i: (i, 0)),
            pl.BlockSpec((_BR, 128), lambda i: (i, 0)),
            pl.BlockSpec((128, 8 * EMBED_DIM), lambda i: (0, 0)),
            pl.BlockSpec((128, 8 * EMBED_DIM), lambda i: (0, 0)),
        ],
        out_specs=pl.BlockSpec((_BR, 8 * EMBED_DIM), lambda i: (i, 0)),
        out_shape=jax.ShapeDtypeStruct((_PR, 8 * EMBED_DIM), jnp.float32),
    )(amp_p, phase_p, sb, cb)


_VPAD = 1000064


def _flatten_table(tab):
    p = jnp.pad(tab.T, ((0, 0), (0, _VPAD - VOCAB)))
    q2 = p.reshape(HARMONIC_BASES * _VPAD // 128, 128)
    q2 = jax.lax.optimization_barrier(q2)
    return q2.reshape(HARMONIC_BASES * _VPAD)


def kernel(x, frequency_amplitudes, frequency_phases, frequencies):
    idx = x.reshape(_T).astype(jnp.int32)
    harm = jnp.tile(jnp.arange(HARMONIC_BASES, dtype=jnp.int32) * _VPAD, _T)
    ilist = jnp.repeat(idx, HARMONIC_BASES) + harm
    flat_a = _flatten_table(frequency_amplitudes)
    flat_p = _flatten_table(frequency_phases)

    a_flat, p_flat = _sc_gather(ilist, flat_a, flat_p)
    amp_p = a_flat.reshape(_PR, 128)
    phase_p = p_flat.reshape(_PR, 128)

    t = jnp.linspace(0.0, 1.0, EMBED_DIM, dtype=jnp.float32)
    theta = (2.0 * math.pi) * frequencies[:, None] * t[None, :]
    eye8 = jnp.eye(8, dtype=jnp.float32)
    sb = jnp.kron(eye8, jnp.sin(theta))
    cb = jnp.kron(eye8, jnp.cos(theta))

    out = _tc_synth(amp_p, phase_p, sb, cb)
    return out.reshape(_B, _S, EMBED_DIM)
